# baseline (speedup 1.0000x reference)
import functools

import jax
import jax.numpy as jnp
import numpy as np
from jax import lax
from jax.experimental import pallas as pl
from jax.experimental.pallas import tpu as pltpu
from jax.experimental.pallas import tpu_sc as plsc

_NC = 2
_NS = 16
_NW = _NC * _NS


def _sc_gather(table, idx, chunk, grp=5):
    rows, cols = table.shape
    total = idx.shape[0]
    n_chunks = total // chunk
    cpw = n_chunks // _NW
    nsup = cpw // grp
    idx2 = idx.reshape(_NW, cpw, chunk)
    mesh = plsc.VectorSubcoreMesh(core_axis_name="c", subcore_axis_name="s")

    @functools.partial(
        pl.kernel,
        mesh=mesh,
        out_type=jax.ShapeDtypeStruct((total, cols), jnp.float32),
        scratch_types=[
            pltpu.VMEM((cpw, chunk), jnp.int32),
            pltpu.VMEM((2, grp, chunk, cols), jnp.float32),
            pltpu.SemaphoreType.DMA,
            pltpu.SemaphoreType.DMA,
        ],
        compiler_params=pltpu.CompilerParams(use_tc_tiling_on_sc=False),
    )
    def gather_kernel(table_hbm, idx_hbm, out_hbm, idx_v, rows_v, sem_g, sem_s):
        wid = lax.axis_index("s") * _NC + lax.axis_index("c")
        base = wid * cpw
        pltpu.sync_copy(idx_hbm.at[wid], idx_v)

        def body(s, carry):
            p = lax.rem(s, 2)

            @pl.when(s >= 2)
            def _drain_prev_stores():
                for j in range(grp):
                    pltpu.make_async_copy(
                        rows_v.at[0, j], out_hbm.at[pl.ds(0, chunk)], sem_s
                    ).wait()

            descs = [
                pltpu.async_copy(
                    table_hbm.at[idx_v.at[s * grp + j]], rows_v.at[p, j], sem_g)
                for j in range(grp)
            ]
            for d in descs:
                d.wait()
            for j in range(grp):
                off = pl.multiple_of((base + s * grp + j) * chunk, 8)
                pltpu.async_copy(rows_v.at[p, j],
                                 out_hbm.at[pl.ds(off, chunk)], sem_s)
            return carry

        lax.fori_loop(0, nsup, body, 0)
        for j in range(2 * grp):
            pltpu.make_async_copy(
                rows_v.at[0, j % grp], out_hbm.at[pl.ds(0, chunk)], sem_s
            ).wait()

    return gather_kernel(table, idx2)


def _tc_body(g_ref, w_ref, a_ref, w2t_ref, b_ref, r20_ref, s20_ref, o_ref,
             *, K, C_IN, C_MID, C_ADD):
    t = g_ref.shape[0]
    ct = C_IN + C_ADD
    width = C_MID * ct
    r20 = r20_ref[...]
    s20 = s20_ref[...]
    pc = jnp.zeros((t, width), jnp.float32)
    for k in range(K):
        gk = g_ref[:, k * C_IN:(k + 1) * C_IN]
        ak = a_ref[:, k * C_ADD:(k + 1) * C_ADD]
        wk = w_ref[:, k * C_MID:(k + 1) * C_MID]
        gcat = jnp.concatenate([gk, ak], axis=1)
        g_rep = jnp.repeat(gcat, C_MID, axis=1)
        w_til = jnp.dot(wk, s20, preferred_element_type=jnp.float32)
        pc = pc + g_rep * w_til
    o_ref[...] = (jnp.dot(pc, w2t_ref[...], preferred_element_type=jnp.float32)
                  + b_ref[...])


def _tc_fused(gathered2, wn2, add2, w2_t, bias2, *, K, C_IN, C_MID,
              C_ADD, OUT_F, tile, interpret=False):
    bn = gathered2.shape[0]
    grid = (bn // tile,)
    ct = C_IN + C_ADD
    width = C_MID * ct
    r20 = jnp.asarray(np.repeat(np.eye(ct, dtype=np.float32), C_MID, axis=1))
    s20 = jnp.asarray(np.tile(np.eye(C_MID, dtype=np.float32), (1, ct)))
    body = functools.partial(_tc_body, K=K, C_IN=C_IN, C_MID=C_MID, C_ADD=C_ADD)
    zero = lambda i: (0, 0)
    return pl.pallas_call(
        body,
        grid=grid,
        in_specs=[
            pl.BlockSpec((tile, K * C_IN), lambda i: (i, 0)),
            pl.BlockSpec((tile, K * C_MID), lambda i: (i, 0)),
            pl.BlockSpec((tile, K * C_ADD), lambda i: (i, 0)),
            pl.BlockSpec((width, OUT_F), zero),
            pl.BlockSpec((1, OUT_F), zero),
            pl.BlockSpec((ct, width), zero),
            pl.BlockSpec((C_MID, width), zero),
        ],
        out_specs=pl.BlockSpec((tile, OUT_F), lambda i: (i, 0)),
        out_shape=jax.ShapeDtypeStruct((bn, OUT_F), jnp.float32),
        interpret=interpret,
    )(gathered2, wn2, add2, w2_t, bias2, r20, s20)


def kernel(input_features, neighbor_inds, weightnet, additional_features,
           linear_weight, linear_bias):
    b, n, c_in = input_features.shape
    _, _, k = neighbor_inds.shape
    c_mid = weightnet.shape[-1]
    c_add = additional_features.shape[-1]
    out_f = linear_weight.shape[0]
    bn = b * n

    table = input_features.reshape(bn, c_in)
    offs = (jnp.arange(b, dtype=neighbor_inds.dtype) * n)[:, None, None]
    idx = (neighbor_inds + offs).reshape(bn * k)

    gathered = _sc_gather(table, idx, chunk=80)

    wn2 = weightnet.reshape(bn, k * c_mid)
    add2 = additional_features.reshape(bn, k * c_add)
    bias2 = linear_bias.reshape(1, out_f)
    w2_t = linear_weight.T

    out = _tc_fused(gathered.reshape(bn, k * c_in), wn2, add2, w2_t,
                    bias2, K=k, C_IN=c_in, C_MID=c_mid, C_ADD=c_add,
                    OUT_F=out_f, tile=2000)
    return out.reshape(b, n, out_f)


# speedup vs baseline: 6.9868x; 6.9868x over previous
import functools

import jax
import jax.numpy as jnp
import numpy as np
from jax import lax
from jax.experimental import pallas as pl
from jax.experimental.pallas import tpu as pltpu
from jax.experimental.pallas import tpu_sc as plsc

_NC = 2
_NS = 16
_NW = _NC * _NS


def _sc_gather(table, idx, chunk, grp=5):
    rows, cols = table.shape
    total = idx.shape[0]
    n_chunks = total // chunk
    cpw = n_chunks // _NW
    nsup = cpw // grp
    idx2 = idx.reshape(_NW, cpw, chunk)
    mesh = plsc.VectorSubcoreMesh(core_axis_name="c", subcore_axis_name="s")

    @functools.partial(
        pl.kernel,
        mesh=mesh,
        out_type=jax.ShapeDtypeStruct((total, cols), jnp.float32),
        scratch_types=[
            pltpu.VMEM((cpw, chunk), jnp.int32),
            pltpu.VMEM((2, grp, chunk, cols), jnp.float32),
            pltpu.SemaphoreType.DMA,
            pltpu.SemaphoreType.DMA,
        ],
        compiler_params=pltpu.CompilerParams(use_tc_tiling_on_sc=False),
    )
    def gather_kernel(table_hbm, idx_hbm, out_hbm, idx_v, rows_v, sem_g, sem_s):
        wid = lax.axis_index("s") * _NC + lax.axis_index("c")
        base = wid * cpw
        pltpu.sync_copy(idx_hbm.at[wid], idx_v)

        def body(s, carry):
            p = lax.rem(s, 2)

            @pl.when(s >= 2)
            def _drain_prev_stores():
                for j in range(grp):
                    pltpu.make_async_copy(
                        rows_v.at[0, j], out_hbm.at[pl.ds(0, chunk)], sem_s
                    ).wait()

            descs = [
                pltpu.async_copy(
                    table_hbm.at[idx_v.at[s * grp + j]], rows_v.at[p, j], sem_g)
                for j in range(grp)
            ]
            for d in descs:
                d.wait()
            for j in range(grp):
                off = pl.multiple_of((base + s * grp + j) * chunk, 8)
                pltpu.async_copy(rows_v.at[p, j],
                                 out_hbm.at[pl.ds(off, chunk)], sem_s)
            return carry

        lax.fori_loop(0, nsup, body, 0)
        for j in range(2 * grp):
            pltpu.make_async_copy(
                rows_v.at[0, j % grp], out_hbm.at[pl.ds(0, chunk)], sem_s
            ).wait()

    return gather_kernel(table, idx2)


def _tc_body(g_ref, w_ref, a_ref, w2t_ref, b_ref, r20_ref, s20_ref, o_ref,
             *, K, C_IN, C_MID, C_ADD):
    t = g_ref.shape[0]
    ct = C_IN + C_ADD
    width = C_MID * ct
    r20 = r20_ref[...]
    s20 = s20_ref[...]
    pc = jnp.zeros((t, width), jnp.float32)
    for k in range(K):
        gk = g_ref[:, k * C_IN:(k + 1) * C_IN]
        ak = a_ref[:, k * C_ADD:(k + 1) * C_ADD]
        wk = w_ref[:, k * C_MID:(k + 1) * C_MID]
        gcat = jnp.concatenate([gk, ak], axis=1)
        g_rep = jnp.dot(gcat, r20, preferred_element_type=jnp.float32)
        w_til = jnp.dot(wk, s20, preferred_element_type=jnp.float32)
        pc = pc + g_rep * w_til
    o_ref[...] = (jnp.dot(pc, w2t_ref[...], preferred_element_type=jnp.float32)
                  + b_ref[...])


def _tc_fused(gathered2, wn2, add2, w2_t, bias2, *, K, C_IN, C_MID,
              C_ADD, OUT_F, tile, interpret=False):
    bn = gathered2.shape[0]
    grid = (bn // tile,)
    ct = C_IN + C_ADD
    width = C_MID * ct
    r20 = jnp.asarray(np.repeat(np.eye(ct, dtype=np.float32), C_MID, axis=1))
    s20 = jnp.asarray(np.tile(np.eye(C_MID, dtype=np.float32), (1, ct)))
    body = functools.partial(_tc_body, K=K, C_IN=C_IN, C_MID=C_MID, C_ADD=C_ADD)
    zero = lambda i: (0, 0)
    return pl.pallas_call(
        body,
        grid=grid,
        in_specs=[
            pl.BlockSpec((tile, K * C_IN), lambda i: (i, 0)),
            pl.BlockSpec((tile, K * C_MID), lambda i: (i, 0)),
            pl.BlockSpec((tile, K * C_ADD), lambda i: (i, 0)),
            pl.BlockSpec((width, OUT_F), zero),
            pl.BlockSpec((1, OUT_F), zero),
            pl.BlockSpec((ct, width), zero),
            pl.BlockSpec((C_MID, width), zero),
        ],
        out_specs=pl.BlockSpec((tile, OUT_F), lambda i: (i, 0)),
        out_shape=jax.ShapeDtypeStruct((bn, OUT_F), jnp.float32),
        interpret=interpret,
    )(gathered2, wn2, add2, w2_t, bias2, r20, s20)


def kernel(input_features, neighbor_inds, weightnet, additional_features,
           linear_weight, linear_bias):
    b, n, c_in = input_features.shape
    _, _, k = neighbor_inds.shape
    c_mid = weightnet.shape[-1]
    c_add = additional_features.shape[-1]
    out_f = linear_weight.shape[0]
    bn = b * n

    table = input_features.reshape(bn, c_in)
    offs = (jnp.arange(b, dtype=neighbor_inds.dtype) * n)[:, None, None]
    idx = (neighbor_inds + offs).reshape(bn * k)

    gathered = _sc_gather(table, idx, chunk=80)

    wn2 = weightnet.reshape(bn, k * c_mid)
    add2 = additional_features.reshape(bn, k * c_add)
    bias2 = linear_bias.reshape(1, out_f)
    w2_t = linear_weight.T

    out = _tc_fused(gathered.reshape(bn, k * c_in), wn2, add2, w2_t,
                    bias2, K=k, C_IN=c_in, C_MID=c_mid, C_ADD=c_add,
                    OUT_F=out_f, tile=1000)
    return out.reshape(b, n, out_f)


# speedup vs baseline: 7.8572x; 1.1246x over previous
import functools

import jax
import jax.numpy as jnp
import numpy as np
from jax import lax
from jax.experimental import pallas as pl
from jax.experimental.pallas import tpu as pltpu
from jax.experimental.pallas import tpu_sc as plsc

_NC = 2
_NS = 16
_NW = _NC * _NS


def _sc_gather(table, idx, chunk, grp=5):
    rows, cols = table.shape
    total = idx.shape[0]
    n_chunks = total // chunk
    cpw = n_chunks // _NW
    nsup = cpw // grp
    idx2 = idx.reshape(_NW, cpw, chunk)
    mesh = plsc.VectorSubcoreMesh(core_axis_name="c", subcore_axis_name="s")

    @functools.partial(
        pl.kernel,
        mesh=mesh,
        out_type=jax.ShapeDtypeStruct((total, cols), jnp.float32),
        scratch_types=[
            pltpu.VMEM((cpw, chunk), jnp.int32),
            pltpu.VMEM((2, grp, chunk, cols), jnp.float32),
            pltpu.SemaphoreType.DMA,
            pltpu.SemaphoreType.DMA,
        ],
        compiler_params=pltpu.CompilerParams(use_tc_tiling_on_sc=False),
    )
    def gather_kernel(table_hbm, idx_hbm, out_hbm, idx_v, rows_v, sem_g, sem_s):
        wid = lax.axis_index("s") * _NC + lax.axis_index("c")
        base = wid * cpw
        pltpu.sync_copy(idx_hbm.at[wid], idx_v)

        def body(s, carry):
            p = lax.rem(s, 2)

            @pl.when(s >= 2)
            def _drain_prev_stores():
                for j in range(grp):
                    pltpu.make_async_copy(
                        rows_v.at[0, j], out_hbm.at[pl.ds(0, chunk)], sem_s
                    ).wait()

            descs = [
                pltpu.async_copy(
                    table_hbm.at[idx_v.at[s * grp + j]], rows_v.at[p, j], sem_g)
                for j in range(grp)
            ]
            for d in descs:
                d.wait()
            for j in range(grp):
                off = pl.multiple_of((base + s * grp + j) * chunk, 8)
                pltpu.async_copy(rows_v.at[p, j],
                                 out_hbm.at[pl.ds(off, chunk)], sem_s)
            return carry

        lax.fori_loop(0, nsup, body, 0)
        for j in range(2 * grp):
            pltpu.make_async_copy(
                rows_v.at[0, j % grp], out_hbm.at[pl.ds(0, chunk)], sem_s
            ).wait()

    return gather_kernel(table, idx2)


def _tc_body(g_ref, w_ref, a_ref, w2t_ref, b_ref, r20_ref, s20_ref, o_ref,
             *, K, C_IN, C_MID, C_ADD):
    t = g_ref.shape[0]
    ct = C_IN + C_ADD
    width = C_MID * ct
    r20 = r20_ref[...]
    s20 = s20_ref[...]
    pc = jnp.zeros((t, width), jnp.float32)
    for k in range(K):
        gk = g_ref[:, k * C_IN:(k + 1) * C_IN]
        ak = a_ref[:, k * C_ADD:(k + 1) * C_ADD]
        wk = w_ref[:, k * C_MID:(k + 1) * C_MID]
        gcat = jnp.concatenate([gk, ak], axis=1)
        g_rep = jnp.dot(gcat, r20, preferred_element_type=jnp.float32)
        w_til = jnp.dot(wk, s20, preferred_element_type=jnp.float32)
        pc = pc + g_rep * w_til
    o_ref[...] = (jnp.dot(pc, w2t_ref[...], preferred_element_type=jnp.float32)
                  + b_ref[...])


def _tc_fused(gathered2, wn2, add2, w2_t, bias2, *, K, C_IN, C_MID,
              C_ADD, OUT_F, tile, interpret=False):
    bn = gathered2.shape[0]
    grid = (bn // tile,)
    ct = C_IN + C_ADD
    width = C_MID * ct
    r20 = jnp.asarray(np.repeat(np.eye(ct, dtype=np.float32), C_MID, axis=1))
    s20 = jnp.asarray(np.tile(np.eye(C_MID, dtype=np.float32), (1, ct)))
    body = functools.partial(_tc_body, K=K, C_IN=C_IN, C_MID=C_MID, C_ADD=C_ADD)
    zero = lambda i: (0, 0)
    return pl.pallas_call(
        body,
        grid=grid,
        in_specs=[
            pl.BlockSpec((tile, K * C_IN), lambda i: (i, 0)),
            pl.BlockSpec((tile, K * C_MID), lambda i: (i, 0)),
            pl.BlockSpec((tile, K * C_ADD), lambda i: (i, 0)),
            pl.BlockSpec((width, OUT_F), zero),
            pl.BlockSpec((1, OUT_F), zero),
            pl.BlockSpec((ct, width), zero),
            pl.BlockSpec((C_MID, width), zero),
        ],
        out_specs=pl.BlockSpec((tile, OUT_F), lambda i: (i, 0)),
        out_shape=jax.ShapeDtypeStruct((bn, OUT_F), jnp.float32),
        interpret=interpret,
    )(gathered2, wn2, add2, w2_t, bias2, r20, s20)


def kernel(input_features, neighbor_inds, weightnet, additional_features,
           linear_weight, linear_bias):
    b, n, c_in = input_features.shape
    _, _, k = neighbor_inds.shape
    c_mid = weightnet.shape[-1]
    c_add = additional_features.shape[-1]
    out_f = linear_weight.shape[0]
    bn = b * n

    table = input_features.reshape(bn, c_in)
    offs = (jnp.arange(b, dtype=neighbor_inds.dtype) * n)[:, None, None]
    idx = (neighbor_inds + offs).reshape(bn * k)

    gathered = _sc_gather(table, idx, chunk=80)

    wn2 = weightnet.reshape(bn, k * c_mid)
    add2 = additional_features.reshape(bn, k * c_add)
    bias2 = linear_bias.reshape(1, out_f)
    w2_t = linear_weight.T

    out = _tc_fused(gathered.reshape(bn, k * c_in), wn2, add2, w2_t,
                    bias2, K=k, C_IN=c_in, C_MID=c_mid, C_ADD=c_add,
                    OUT_F=out_f, tile=4000)
    return out.reshape(b, n, out_f)


# speedup vs baseline: 8.0490x; 1.0244x over previous
import functools

import jax
import jax.numpy as jnp
import numpy as np
from jax import lax
from jax.experimental import pallas as pl
from jax.experimental.pallas import tpu as pltpu
from jax.experimental.pallas import tpu_sc as plsc

_NC = 2
_NS = 16
_NW = _NC * _NS


def _sc_gather(table, idx, chunk, grp=5):
    rows, cols = table.shape
    total = idx.shape[0]
    n_chunks = total // chunk
    cpw = n_chunks // _NW
    nsup = cpw // grp
    idx2 = idx.reshape(_NW, cpw, chunk)
    mesh = plsc.VectorSubcoreMesh(core_axis_name="c", subcore_axis_name="s")

    @functools.partial(
        pl.kernel,
        mesh=mesh,
        out_type=jax.ShapeDtypeStruct((total, cols), jnp.float32),
        scratch_types=[
            pltpu.VMEM((cpw, chunk), jnp.int32),
            pltpu.VMEM((2, grp, chunk, cols), jnp.float32),
            pltpu.SemaphoreType.DMA,
            pltpu.SemaphoreType.DMA,
        ],
        compiler_params=pltpu.CompilerParams(use_tc_tiling_on_sc=False),
    )
    def gather_kernel(table_hbm, idx_hbm, out_hbm, idx_v, rows_v, sem_g, sem_s):
        wid = lax.axis_index("s") * _NC + lax.axis_index("c")
        base = wid * cpw
        pltpu.sync_copy(idx_hbm.at[wid], idx_v)

        def body(s, carry):
            p = lax.rem(s, 2)

            @pl.when(s >= 2)
            def _drain_prev_stores():
                for j in range(grp):
                    pltpu.make_async_copy(
                        rows_v.at[0, j], out_hbm.at[pl.ds(0, chunk)], sem_s
                    ).wait()

            descs = [
                pltpu.async_copy(
                    table_hbm.at[idx_v.at[s * grp + j]], rows_v.at[p, j], sem_g)
                for j in range(grp)
            ]
            for d in descs:
                d.wait()
            for j in range(grp):
                off = pl.multiple_of((base + s * grp + j) * chunk, 8)
                pltpu.async_copy(rows_v.at[p, j],
                                 out_hbm.at[pl.ds(off, chunk)], sem_s)
            return carry

        lax.fori_loop(0, nsup, body, 0)
        for j in range(2 * grp):
            pltpu.make_async_copy(
                rows_v.at[0, j % grp], out_hbm.at[pl.ds(0, chunk)], sem_s
            ).wait()

    return gather_kernel(table, idx2)


def _tc_body(g_ref, w_ref, a_ref, w2t_ref, b_ref, r20_ref, s20_ref, o_ref,
             *, K, C_IN, C_MID, C_ADD):
    t = g_ref.shape[0]
    ct = C_IN + C_ADD
    width = C_MID * ct
    r20 = r20_ref[...]
    s20 = s20_ref[...]
    pc = jnp.zeros((t, width), jnp.float32)
    for k in range(K):
        gk = g_ref[:, k * C_IN:(k + 1) * C_IN]
        ak = a_ref[:, k * C_ADD:(k + 1) * C_ADD]
        wk = w_ref[:, k * C_MID:(k + 1) * C_MID]
        gcat = jnp.concatenate([gk, ak], axis=1)
        g_rep = jnp.dot(gcat, r20, preferred_element_type=jnp.float32)
        w_til = jnp.dot(wk, s20, preferred_element_type=jnp.float32)
        pc = pc + g_rep * w_til
    o_ref[...] = (jnp.dot(pc, w2t_ref[...], preferred_element_type=jnp.float32)
                  + b_ref[...])


def _tc_fused(gathered2, wn2, add2, w2_t, bias2, *, K, C_IN, C_MID,
              C_ADD, OUT_F, tile, interpret=False):
    bn = gathered2.shape[0]
    grid = (bn // tile,)
    ct = C_IN + C_ADD
    width = C_MID * ct
    r20 = jnp.asarray(np.repeat(np.eye(ct, dtype=np.float32), C_MID, axis=1))
    s20 = jnp.asarray(np.tile(np.eye(C_MID, dtype=np.float32), (1, ct)))
    body = functools.partial(_tc_body, K=K, C_IN=C_IN, C_MID=C_MID, C_ADD=C_ADD)
    zero = lambda i: (0, 0)
    return pl.pallas_call(
        body,
        grid=grid,
        in_specs=[
            pl.BlockSpec((tile, K * C_IN), lambda i: (i, 0)),
            pl.BlockSpec((tile, K * C_MID), lambda i: (i, 0)),
            pl.BlockSpec((tile, K * C_ADD), lambda i: (i, 0)),
            pl.BlockSpec((width, OUT_F), zero),
            pl.BlockSpec((1, OUT_F), zero),
            pl.BlockSpec((ct, width), zero),
            pl.BlockSpec((C_MID, width), zero),
        ],
        out_specs=pl.BlockSpec((tile, OUT_F), lambda i: (i, 0)),
        out_shape=jax.ShapeDtypeStruct((bn, OUT_F), jnp.float32),
        interpret=interpret,
    )(gathered2, wn2, add2, w2_t, bias2, r20, s20)


def kernel(input_features, neighbor_inds, weightnet, additional_features,
           linear_weight, linear_bias):
    b, n, c_in = input_features.shape
    _, _, k = neighbor_inds.shape
    c_mid = weightnet.shape[-1]
    c_add = additional_features.shape[-1]
    out_f = linear_weight.shape[0]
    bn = b * n

    table = input_features.reshape(bn, c_in)
    offs = (jnp.arange(b, dtype=neighbor_inds.dtype) * n)[:, None, None]
    idx = (neighbor_inds + offs).reshape(bn * k)

    gathered = _sc_gather(table, idx, chunk=80)

    wn2 = weightnet.reshape(bn, k * c_mid)
    add2 = additional_features.reshape(bn, k * c_add)
    bias2 = linear_bias.reshape(1, out_f)
    w2_t = linear_weight.T

    out = _tc_fused(gathered.reshape(bn, k * c_in), wn2, add2, w2_t,
                    bias2, K=k, C_IN=c_in, C_MID=c_mid, C_ADD=c_add,
                    OUT_F=out_f, tile=2000)
    return out.reshape(b, n, out_f)
